# baseline (speedup 1.0000x reference)
import functools

import jax
import jax.numpy as jnp
from jax import lax
from jax.experimental import pallas as pl
from jax.experimental.pallas import tpu as pltpu

EPSILON = 1e-6
MATCH_THRESHOLD = 0.5
RADIUS_THRESHOLD = 0.05

NB = 512
SCHUNK = 4096


def _fused_kernel(hid_ref, wcat_ref, wbd_ref, b13_ref, b24_ref, bel_ref,
                  andm_ref, orm_ref, obsb_ref, slots_ref, simsout_ref,
                  wallT_ref, angsT_ref):
    i = pl.program_id(0)
    S = bel_ref.shape[0]

    @pl.when(i == 0)
    def _init():
        wallT_ref[...] = wcat_ref[...].T
        bel = bel_ref[...]
        norm = jnp.sqrt(jnp.sum(bel * bel, axis=1, keepdims=True))
        angsT_ref[...] = (bel / jnp.maximum(norm, EPSILON)).astype(jnp.bfloat16).T

    hb = hid_ref[...].astype(jnp.bfloat16)
    acc = jnp.dot(hb, wallT_ref[...], preferred_element_type=jnp.float32)
    obs = acc[:, :256]
    h13 = jnp.maximum(acc[:, 256:] + b13_ref[...], 0.0)
    gl = lax.dot_general(h13.astype(jnp.bfloat16), wbd_ref[...],
                         (((1,), (1,)), ((), ())),
                         preferred_element_type=jnp.float32) + b24_ref[...]
    gate = jax.nn.sigmoid(gl[:, 0:1])
    prec = jax.nn.softplus(gl[:, 1:2])
    gp = gate * prec

    onorm = jnp.sqrt(jnp.sum(obs * obs, axis=1, keepdims=True))
    rinv = 1.0 / jnp.maximum(onorm, EPSILON)
    obsb_ref[...] = obs * (rinv * gp)

    obs_bf = obs.astype(jnp.bfloat16)
    pmax = None
    for j in range(S // SCHUNK):
        sl = pl.ds(j * SCHUNK, SCHUNK)
        raw = jnp.dot(obs_bf, angsT_ref[:, sl],
                      preferred_element_type=jnp.float32)
        b = lax.bitcast_convert_type(raw, jnp.int32)
        packed = (b & andm_ref[:, sl]) | orm_ref[:, sl]
        m = jnp.max(lax.bitcast_convert_type(packed, jnp.float32), axis=1)
        pmax = m if pmax is None else jnp.maximum(pmax, m)
    pbest = lax.bitcast_convert_type(pmax, jnp.int32)
    bidx = (S - 1) - (pbest & jnp.int32(8191))
    bestv = lax.bitcast_convert_type(pbest & jnp.int32(-8192),
                                     jnp.float32) * rinv[:, 0]
    matched = (gp[:, 0] > RADIUS_THRESHOLD) & (bestv > MATCH_THRESHOLD)
    slots_ref[0, 0, :] = jnp.where(matched, bidx, -1).astype(jnp.int32)
    simsout_ref[0, 0, :] = jnp.where(matched, bestv, 0.0)


@functools.partial(jax.jit, static_argnames=())
def kernel(hidden, beliefs, active_mask, W_obs, w1, b1, w2, b2, w3, b3, w4, b4):
    B, T, H = hidden.shape
    D = W_obs.shape[0]
    Hq = w1.shape[0]
    S = beliefs.shape[0]
    N = B * T
    nblk = N // NB

    hid2d = hidden.reshape(N, H)
    wcat = jnp.concatenate([W_obs, w1, w3], axis=0).astype(jnp.bfloat16)
    wbd = jnp.zeros((2, 2 * Hq), jnp.float32)
    wbd = wbd.at[0, :Hq].set(w2[0]).at[1, Hq:].set(w4[0]).astype(jnp.bfloat16)
    b13 = jnp.concatenate([b1, b3]).reshape(1, 2 * Hq).astype(jnp.float32)
    b24 = jnp.concatenate([b2, b4]).reshape(1, 2).astype(jnp.float32)
    revcol = (S - 1 - jnp.arange(S, dtype=jnp.int32)).reshape(1, S)
    andm = jnp.where(active_mask, jnp.int32(-8192), jnp.int32(0)).reshape(1, S)
    orm = revcol | jnp.where(active_mask, jnp.int32(0),
                             jnp.int32(-2147483648)).reshape(1, S)

    grid = (nblk,)
    obsb, slots3, sims3 = pl.pallas_call(
        _fused_kernel,
        grid=grid,
        in_specs=[
            pl.BlockSpec((NB, H), lambda i: (i, 0)),
            pl.BlockSpec((D + 2 * Hq, H), lambda i: (0, 0)),
            pl.BlockSpec((2, 2 * Hq), lambda i: (0, 0)),
            pl.BlockSpec((1, 2 * Hq), lambda i: (0, 0)),
            pl.BlockSpec((1, 2), lambda i: (0, 0)),
            pl.BlockSpec((S, D), lambda i: (0, 0)),
            pl.BlockSpec((1, S), lambda i: (0, 0)),
            pl.BlockSpec((1, S), lambda i: (0, 0)),
        ],
        out_specs=[
            pl.BlockSpec((NB, D), lambda i: (i, 0)),
            pl.BlockSpec((1, 1, NB), lambda i: (i, 0, 0)),
            pl.BlockSpec((1, 1, NB), lambda i: (i, 0, 0)),
        ],
        out_shape=[
            jax.ShapeDtypeStruct((N, D), jnp.float32),
            jax.ShapeDtypeStruct((nblk, 1, NB), jnp.int32),
            jax.ShapeDtypeStruct((nblk, 1, NB), jnp.float32),
        ],
        scratch_shapes=[
            pltpu.VMEM((H, D + 2 * Hq), jnp.bfloat16),
            pltpu.VMEM((D, S), jnp.bfloat16),
        ],
    )(hid2d, wcat, wbd, b13, b24, beliefs, andm, orm)

    return (obsb.reshape(B, T, D), slots3.reshape(N), sims3.reshape(N))


# speedup vs baseline: 1.0026x; 1.0026x over previous
import functools

import jax
import jax.numpy as jnp
from jax import lax
from jax.experimental import pallas as pl
from jax.experimental.pallas import tpu as pltpu

EPSILON = 1e-6
MATCH_THRESHOLD = 0.5
RADIUS_THRESHOLD = 0.05

NB = 512
SCHUNK = 2048


def _fused_kernel(hid_ref, wcat_ref, wbd_ref, b13_ref, b24_ref, bel_ref,
                  andm_ref, orm_ref, obsb_ref, slots_ref, simsout_ref,
                  wallT_ref, angsT_ref):
    i = pl.program_id(0)
    S = bel_ref.shape[0]

    @pl.when(i == 0)
    def _init():
        wallT_ref[...] = wcat_ref[...].T
        bel = bel_ref[...]
        norm = jnp.sqrt(jnp.sum(bel * bel, axis=1, keepdims=True))
        angsT_ref[...] = (bel / jnp.maximum(norm, EPSILON)).astype(jnp.bfloat16).T

    hb = hid_ref[...].astype(jnp.bfloat16)
    acc = jnp.dot(hb, wallT_ref[...], preferred_element_type=jnp.float32)
    obs = acc[:, :256]
    h13 = jnp.maximum(acc[:, 256:] + b13_ref[...], 0.0)
    gl = lax.dot_general(h13.astype(jnp.bfloat16), wbd_ref[...],
                         (((1,), (1,)), ((), ())),
                         preferred_element_type=jnp.float32) + b24_ref[...]
    gate = jax.nn.sigmoid(gl[:, 0:1])
    prec = jax.nn.softplus(gl[:, 1:2])
    gp = gate * prec

    onorm = jnp.sqrt(jnp.sum(obs * obs, axis=1, keepdims=True))
    rinv = 1.0 / jnp.maximum(onorm, EPSILON)
    obsb_ref[...] = obs * (rinv * gp)

    obs_bf = obs.astype(jnp.bfloat16)
    pmax = None
    for j in range(S // SCHUNK):
        sl = pl.ds(j * SCHUNK, SCHUNK)
        raw = jnp.dot(obs_bf, angsT_ref[:, sl],
                      preferred_element_type=jnp.float32)
        b = lax.bitcast_convert_type(raw, jnp.int32)
        packed = (b & andm_ref[:, sl]) | orm_ref[:, sl]
        m = jnp.max(lax.bitcast_convert_type(packed, jnp.float32), axis=1)
        pmax = m if pmax is None else jnp.maximum(pmax, m)
    pbest = lax.bitcast_convert_type(pmax, jnp.int32)
    bidx = (S - 1) - (pbest & jnp.int32(8191))
    bestv = lax.bitcast_convert_type(pbest & jnp.int32(-8192),
                                     jnp.float32) * rinv[:, 0]
    matched = (gp[:, 0] > RADIUS_THRESHOLD) & (bestv > MATCH_THRESHOLD)
    slots_ref[0, 0, :] = jnp.where(matched, bidx, -1).astype(jnp.int32)
    simsout_ref[0, 0, :] = jnp.where(matched, bestv, 0.0)


@functools.partial(jax.jit, static_argnames=())
def kernel(hidden, beliefs, active_mask, W_obs, w1, b1, w2, b2, w3, b3, w4, b4):
    B, T, H = hidden.shape
    D = W_obs.shape[0]
    Hq = w1.shape[0]
    S = beliefs.shape[0]
    N = B * T
    nblk = N // NB

    hid2d = hidden.reshape(N, H)
    wcat = jnp.concatenate([W_obs, w1, w3], axis=0).astype(jnp.bfloat16)
    wbd = jnp.zeros((2, 2 * Hq), jnp.float32)
    wbd = wbd.at[0, :Hq].set(w2[0]).at[1, Hq:].set(w4[0]).astype(jnp.bfloat16)
    b13 = jnp.concatenate([b1, b3]).reshape(1, 2 * Hq).astype(jnp.float32)
    b24 = jnp.concatenate([b2, b4]).reshape(1, 2).astype(jnp.float32)
    revcol = (S - 1 - jnp.arange(S, dtype=jnp.int32)).reshape(1, S)
    andm = jnp.where(active_mask, jnp.int32(-8192), jnp.int32(0)).reshape(1, S)
    orm = revcol | jnp.where(active_mask, jnp.int32(0),
                             jnp.int32(-2147483648)).reshape(1, S)

    grid = (nblk,)
    obsb, slots3, sims3 = pl.pallas_call(
        _fused_kernel,
        grid=grid,
        in_specs=[
            pl.BlockSpec((NB, H), lambda i: (i, 0)),
            pl.BlockSpec((D + 2 * Hq, H), lambda i: (0, 0)),
            pl.BlockSpec((2, 2 * Hq), lambda i: (0, 0)),
            pl.BlockSpec((1, 2 * Hq), lambda i: (0, 0)),
            pl.BlockSpec((1, 2), lambda i: (0, 0)),
            pl.BlockSpec((S, D), lambda i: (0, 0)),
            pl.BlockSpec((1, S), lambda i: (0, 0)),
            pl.BlockSpec((1, S), lambda i: (0, 0)),
        ],
        out_specs=[
            pl.BlockSpec((NB, D), lambda i: (i, 0)),
            pl.BlockSpec((1, 1, NB), lambda i: (i, 0, 0)),
            pl.BlockSpec((1, 1, NB), lambda i: (i, 0, 0)),
        ],
        out_shape=[
            jax.ShapeDtypeStruct((N, D), jnp.float32),
            jax.ShapeDtypeStruct((nblk, 1, NB), jnp.int32),
            jax.ShapeDtypeStruct((nblk, 1, NB), jnp.float32),
        ],
        scratch_shapes=[
            pltpu.VMEM((H, D + 2 * Hq), jnp.bfloat16),
            pltpu.VMEM((D, S), jnp.bfloat16),
        ],
    )(hid2d, wcat, wbd, b13, b24, beliefs, andm, orm)

    return (obsb.reshape(B, T, D), slots3.reshape(N), sims3.reshape(N))
